# baseline (speedup 1.0000x reference)
import functools

import jax
import jax.numpy as jnp
import numpy as np
from jax import lax
from jax.experimental import pallas as pl
from jax.experimental.pallas import tpu as pltpu
from jax.experimental.pallas import tpu_sc as plsc

VOCAB = 100000
EMB = 128
NLAB = 1000
NLAB_PAD = 1024
B = 4096
L = 200

NC, NS, LANES = 2, 16, 16
NW = NC * NS
BPW = B // NW
IDS_PW = BPW * L
NV = EMB // LANES
G1 = 128
G2 = L - G1


CH = 128
NCH = IDS_PW // CH


def _sc_sum_body(ids_hbm, emb_hbm, out_hbm, idx_v, dst_v, rows_v, zs_v, acc_sh,
                 sem0, sem1):
    c = lax.axis_index("c")
    s = lax.axis_index("s")
    wid = s * NC + c
    base = wid * BPW
    slot0 = s * BPW
    pltpu.sync_copy(ids_hbm.at[pl.ds(base * L, IDS_PW)], idx_v)
    sems = (sem0, sem1)

    PERIOD = 25
    QSTEP = CH * PERIOD // L
    lane = lax.iota(jnp.int32, LANES)

    def dst_body(p, carry):
        d0 = slot0 + QSTEP * p
        for u in range(PERIOD):
            ch = p * PERIOD + u
            for sub in range(CH // LANES):
                f0 = u * CH + sub * LANES
                qb = f0 // L
                rem = f0 % L
                bump = jnp.where(lane >= (L - rem), 1, 0) if rem + LANES > L \
                    else jnp.zeros((LANES,), jnp.int32)
                dst_v[ch, pl.ds(sub * LANES, LANES)] = d0 + qb + bump
        return carry

    lax.fori_loop(0, NCH // PERIOD, dst_body, 0)

    zv = jnp.zeros((LANES,), jnp.float32)

    def zero_body(r, carry):
        for sub in range(NV):
            zs_v[r, pl.ds(sub * LANES, LANES)] = zv
        return carry

    lax.fori_loop(0, BPW, zero_body, 0)
    pltpu.sync_copy(zs_v, acc_sh.at[pl.ds(slot0, BPW)])

    def fire(ch, buf):
        pltpu.make_async_copy(
            emb_hbm.at[idx_v.at[pl.ds(ch * CH, CH)]],
            rows_v.at[buf], sems[buf]).start()

    def wait(buf):
        pltpu.make_async_copy(
            emb_hbm.at[idx_v.at[pl.ds(0, CH)]],
            rows_v.at[buf], sems[buf]).wait()

    fire(0, 0)
    fire(1, 1)

    def ch_body(g, carry):
        ch0 = 2 * g
        for buf in range(2):
            ch = ch0 + buf
            wait(buf)
            pltpu.sync_copy(rows_v.at[buf], acc_sh.at[dst_v.at[ch]], add=True)

            @pl.when(ch + 2 < NCH)
            def _(buf=buf, ch=ch):
                fire(ch + 2, buf)
        return carry

    lax.fori_loop(0, NCH // 2, ch_body, 0)
    pltpu.sync_copy(acc_sh.at[pl.ds(slot0, BPW)], out_hbm.at[pl.ds(base, BPW)])


_sc_sum = functools.partial(
    pl.kernel,
    out_type=jax.ShapeDtypeStruct((B, EMB), jnp.float32),
    mesh=plsc.VectorSubcoreMesh(core_axis_name="c", subcore_axis_name="s"),
    scratch_types=[
        pltpu.VMEM((IDS_PW,), jnp.int32),
        pltpu.VMEM((NCH, CH), jnp.int32),
        pltpu.VMEM((2, CH, EMB), jnp.float32),
        pltpu.VMEM((BPW, EMB), jnp.float32),
        pltpu.VMEM_SHARED((NS * BPW, EMB), jnp.float32),
        pltpu.SemaphoreType.DMA,
        pltpu.SemaphoreType.DMA,
    ],
)(_sc_sum_body)


def _tc_fc_body(sum_ref, len_ref, w_ref, b_ref, out_ref):
    inv = 1.0 / jnp.maximum(len_ref[...], 1.0)
    mean = sum_ref[...] * inv
    out_ref[...] = (
        jnp.dot(mean, w_ref[...], preferred_element_type=jnp.float32)
        + b_ref[...])


def kernel(ids, lengths, emb, W, b):
    ids_flat = ids.reshape(-1).astype(jnp.int32)
    summed = _sc_sum(ids_flat, emb)

    lenf = lengths.astype(jnp.float32).reshape(B, 1)
    bp = b.reshape(1, NLAB)

    BT = 512
    out = pl.pallas_call(
        _tc_fc_body,
        grid=(B // BT,),
        in_specs=[
            pl.BlockSpec((BT, EMB), lambda i: (i, 0)),
            pl.BlockSpec((BT, 1), lambda i: (i, 0)),
            pl.BlockSpec((EMB, NLAB), lambda i: (0, 0)),
            pl.BlockSpec((1, NLAB), lambda i: (0, 0)),
        ],
        out_specs=pl.BlockSpec((BT, NLAB), lambda i: (i, 0)),
        out_shape=jax.ShapeDtypeStruct((B, NLAB), jnp.float32),
    )(summed, lenf, W, bp)
    return out


# speedup vs baseline: 1.1959x; 1.1959x over previous
import functools

import jax
import jax.numpy as jnp
from jax import lax
from jax.experimental import pallas as pl
from jax.experimental.pallas import tpu as pltpu
from jax.experimental.pallas import tpu_sc as plsc

VOCAB = 100000
EMB = 128
NLAB = 1000
B = 4096
L = 200

NC, NS, LANES = 2, 16, 16
NW = NC * NS
BPW = B // NW
IDS_PW = BPW * L
NV = EMB // LANES
G1 = 128
G2 = L - G1
UNROLL = 8

KD = 48
KV = BPW - KD
CH = 128
NCHD = KD * L // CH
PERIOD = 25
QSTEP = CH * PERIOD // L
NRING = 4


def _sc_sum_body(ids_hbm, emb_hbm, out_hbm, idx_v, dstd_v, rows_v, gbuf_v,
                 ring_v, acc_sh, semv0, semv1, semg0, semg1, sema0, sema1,
                 semo):
    c = lax.axis_index("c")
    s = lax.axis_index("s")
    wid = s * NC + c
    base = wid * BPW
    slot0 = s * KD
    pltpu.sync_copy(ids_hbm.at[pl.ds(base * L, IDS_PW)], idx_v)
    semsv = (semv0, semv1)
    semsg = (semg0, semg1)
    semsa = (sema0, sema1)

    lane = lax.iota(jnp.int32, LANES)

    def dst_body(p, carry):
        d0 = slot0 + QSTEP * p
        for u in range(PERIOD):
            ch = p * PERIOD + u
            for sub in range(CH // LANES):
                f0 = u * CH + sub * LANES
                qb = f0 // L
                rem = f0 % L
                bump = jnp.where(lane >= (L - rem), 1, 0) if rem + LANES > L \
                    else jnp.zeros((LANES,), jnp.int32)
                dstd_v[ch, pl.ds(sub * LANES, LANES)] = d0 + qb + bump
        return carry

    lax.fori_loop(0, NCHD // PERIOD, dst_body, 0)

    zv = jnp.zeros((LANES,), jnp.float32)

    def zero_body(r, carry):
        for sub in range(NV):
            gbuf_v[0, r, pl.ds(sub * LANES, LANES)] = zv
        return carry

    lax.fori_loop(0, KD, zero_body, 0)
    pltpu.sync_copy(gbuf_v.at[0, pl.ds(0, KD)], acc_sh.at[pl.ds(slot0, KD)])

    def vfire(r, buf):
        off = r * L
        pltpu.make_async_copy(
            emb_hbm.at[idx_v.at[pl.ds(off, G1)]],
            rows_v.at[buf, pl.ds(0, G1)], semsv[buf]).start()
        pltpu.make_async_copy(
            emb_hbm.at[idx_v.at[pl.ds(off + G1, G2)]],
            rows_v.at[buf, pl.ds(G1, G2)], semsv[buf]).start()

    def vwait(buf):
        pltpu.make_async_copy(
            emb_hbm.at[idx_v.at[pl.ds(0, G1)]],
            rows_v.at[buf, pl.ds(0, G1)], semsv[buf]).wait()
        pltpu.make_async_copy(
            emb_hbm.at[idx_v.at[pl.ds(0, G2)]],
            rows_v.at[buf, pl.ds(G1, G2)], semsv[buf]).wait()

    zeros = tuple(jnp.zeros((LANES,), jnp.float32) for _ in range(NV))

    def accum(buf, m):
        def acc_body(t, acc):
            j = t * UNROLL
            for u in range(UNROLL):
                acc = tuple(
                    acc[k] + rows_v[buf, j + u, pl.ds(k * LANES, LANES)]
                    for k in range(NV))
            return acc

        acc = lax.fori_loop(0, L // UNROLL, acc_body, zeros)
        for k in range(NV):
            ring_v[m, pl.ds(k * LANES, LANES)] = acc[k]

    def out_fire(r, m):
        pltpu.make_async_copy(
            ring_v.at[pl.ds(m, 1)], out_hbm.at[pl.ds(base + r, 1)],
            semo).start()

    def out_drain(m):
        pltpu.make_async_copy(
            ring_v.at[pl.ds(m, 1)], out_hbm.at[pl.ds(base, 1)], semo).wait()

    def dfire(ch, buf):
        pltpu.make_async_copy(
            emb_hbm.at[idx_v.at[pl.ds(KV * L + ch * CH, CH)]],
            gbuf_v.at[buf], semsg[buf]).start()

    def dwait_gather(buf):
        pltpu.make_async_copy(
            emb_hbm.at[idx_v.at[pl.ds(0, CH)]],
            gbuf_v.at[buf], semsg[buf]).wait()

    def dadd_start(ch, buf):
        pltpu.make_async_copy(
            gbuf_v.at[buf], acc_sh.at[dstd_v.at[ch]],
            semsa[buf]).start(add=True)

    def dadd_wait(ch, buf):
        pltpu.make_async_copy(
            gbuf_v.at[buf], acc_sh.at[dstd_v.at[ch]], semsa[buf]).wait()

    vfire(0, 0)
    vfire(1, 1)
    dfire(0, 0)
    dfire(1, 1)

    def pair_body(g, carry):
        sl0 = 2 * g
        for buf in range(2):
            sl = sl0 + buf
            m = sl % NRING

            @pl.when(sl < NCHD)
            def _(buf=buf, sl=sl):
                dwait_gather(buf)
                dadd_start(sl, buf)

            @pl.when(sl >= NRING)
            def _(m=m):
                out_drain(m)

            vwait(buf)
            accum(buf, m)
            out_fire(sl, m)

            @pl.when(sl + 2 < KV)
            def _(buf=buf, sl=sl):
                vfire(sl + 2, buf)

            @pl.when(sl < NCHD)
            def _(buf=buf, sl=sl):
                dadd_wait(sl, buf)

                @pl.when(sl + 2 < NCHD)
                def _(buf=buf, sl=sl):
                    dfire(sl + 2, buf)
        return carry

    lax.fori_loop(0, KV // 2, pair_body, 0)
    for _ in range(NRING):
        out_drain(0)
    pltpu.sync_copy(acc_sh.at[pl.ds(slot0, KD)],
                    out_hbm.at[pl.ds(base + KV, KD)])


_sc_sum = functools.partial(
    pl.kernel,
    out_type=jax.ShapeDtypeStruct((B, EMB), jnp.float32),
    mesh=plsc.VectorSubcoreMesh(core_axis_name="c", subcore_axis_name="s"),
    scratch_types=[
        pltpu.VMEM((IDS_PW,), jnp.int32),
        pltpu.VMEM((NCHD, CH), jnp.int32),
        pltpu.VMEM((2, L, EMB), jnp.float32),
        pltpu.VMEM((2, CH, EMB), jnp.float32),
        pltpu.VMEM((NRING, EMB), jnp.float32),
        pltpu.VMEM_SHARED((NS * KD, EMB), jnp.float32),
        pltpu.SemaphoreType.DMA,
        pltpu.SemaphoreType.DMA,
        pltpu.SemaphoreType.DMA,
        pltpu.SemaphoreType.DMA,
        pltpu.SemaphoreType.DMA,
        pltpu.SemaphoreType.DMA,
        pltpu.SemaphoreType.DMA,
    ],
)(_sc_sum_body)


def _tc_fc_body(sum_ref, len_ref, w_ref, b_ref, out_ref):
    inv = 1.0 / jnp.maximum(len_ref[...], 1.0)
    mean = sum_ref[...] * inv
    out_ref[...] = (
        jnp.dot(mean, w_ref[...], preferred_element_type=jnp.float32)
        + b_ref[...])


def kernel(ids, lengths, emb, W, b):
    ids_flat = ids.reshape(-1).astype(jnp.int32)
    summed = _sc_sum(ids_flat, emb)

    lenf = lengths.astype(jnp.float32).reshape(B, 1)
    bp = b.reshape(1, NLAB)

    BT = 512
    out = pl.pallas_call(
        _tc_fc_body,
        grid=(B // BT,),
        in_specs=[
            pl.BlockSpec((BT, EMB), lambda i: (i, 0)),
            pl.BlockSpec((BT, 1), lambda i: (i, 0)),
            pl.BlockSpec((EMB, NLAB), lambda i: (0, 0)),
            pl.BlockSpec((1, NLAB), lambda i: (0, 0)),
        ],
        out_specs=pl.BlockSpec((BT, NLAB), lambda i: (i, 0)),
        out_shape=jax.ShapeDtypeStruct((B, NLAB), jnp.float32),
    )(summed, lenf, W, bp)
    return out


# speedup vs baseline: 1.3375x; 1.1184x over previous
import functools

import jax
import jax.numpy as jnp
from jax import lax
from jax.experimental import pallas as pl
from jax.experimental.pallas import tpu as pltpu
from jax.experimental.pallas import tpu_sc as plsc

VOCAB = 100000
EMB = 128
NLAB = 1000
NLAB_PAD = 1024
B = 4096
L = 200

NC, NS, LANES = 2, 16, 16
NW = NC * NS
BPW = B // NW
IDS_PW = BPW * L
NV = EMB // LANES
G1 = 128
G2 = L - G1


UNROLL = 8


def _sc_sum_body(ids_hbm, emb_hbm, out_hbm, idx_v, rows_v, sums_v, sem0, sem1):
    c = lax.axis_index("c")
    s = lax.axis_index("s")
    wid = s * NC + c
    base = wid * BPW
    pltpu.sync_copy(ids_hbm.at[pl.ds(base * L, IDS_PW)], idx_v)
    sems = (sem0, sem1)

    def fire(r, buf):
        off = r * L
        pltpu.make_async_copy(
            emb_hbm.at[idx_v.at[pl.ds(off, G1)]],
            rows_v.at[buf, pl.ds(0, G1)], sems[buf]).start()
        pltpu.make_async_copy(
            emb_hbm.at[idx_v.at[pl.ds(off + G1, G2)]],
            rows_v.at[buf, pl.ds(G1, G2)], sems[buf]).start()

    def wait(buf):
        pltpu.make_async_copy(
            emb_hbm.at[idx_v.at[pl.ds(0, G1)]],
            rows_v.at[buf, pl.ds(0, G1)], sems[buf]).wait()
        pltpu.make_async_copy(
            emb_hbm.at[idx_v.at[pl.ds(0, G2)]],
            rows_v.at[buf, pl.ds(G1, G2)], sems[buf]).wait()

    zeros = tuple(jnp.zeros((LANES,), jnp.float32) for _ in range(NV))

    def accum(buf, r):
        def acc_body(t, acc):
            j = t * UNROLL
            for u in range(UNROLL):
                acc = tuple(
                    acc[k] + rows_v[buf, j + u, pl.ds(k * LANES, LANES)]
                    for k in range(NV))
            return acc

        acc = lax.fori_loop(0, L // UNROLL, acc_body, zeros)
        for k in range(NV):
            sums_v[r, pl.ds(k * LANES, LANES)] = acc[k]

    fire(0, 0)
    fire(1, 1)

    def pair_body(g, carry):
        r0 = 2 * g
        for buf in range(2):
            r = r0 + buf
            wait(buf)
            accum(buf, r)

            @pl.when(r + 2 < BPW)
            def _(buf=buf, r=r):
                fire(r + 2, buf)
        return carry

    lax.fori_loop(0, BPW // 2, pair_body, 0)
    pltpu.sync_copy(sums_v, out_hbm.at[pl.ds(base, BPW)])


_sc_sum = functools.partial(
    pl.kernel,
    out_type=jax.ShapeDtypeStruct((B, EMB), jnp.float32),
    mesh=plsc.VectorSubcoreMesh(core_axis_name="c", subcore_axis_name="s"),
    scratch_types=[
        pltpu.VMEM((IDS_PW,), jnp.int32),
        pltpu.VMEM((2, L, EMB), jnp.float32),
        pltpu.VMEM((BPW, EMB), jnp.float32),
        pltpu.SemaphoreType.DMA,
        pltpu.SemaphoreType.DMA,
    ],
)(_sc_sum_body)


def _tc_fc_body(sum_ref, len_ref, w_ref, b_ref, out_ref):
    inv = 1.0 / jnp.maximum(len_ref[...], 1.0)
    mean = sum_ref[...] * inv
    out_ref[...] = (
        jnp.dot(mean, w_ref[...], preferred_element_type=jnp.float32)
        + b_ref[...])


def kernel(ids, lengths, emb, W, b):
    ids_flat = ids.reshape(-1).astype(jnp.int32)
    summed = _sc_sum(ids_flat, emb)

    lenf = lengths.astype(jnp.float32).reshape(B, 1)
    bp = b.reshape(1, NLAB)

    BT = 512
    out = pl.pallas_call(
        _tc_fc_body,
        grid=(B // BT,),
        in_specs=[
            pl.BlockSpec((BT, EMB), lambda i: (i, 0)),
            pl.BlockSpec((BT, 1), lambda i: (i, 0)),
            pl.BlockSpec((EMB, NLAB), lambda i: (0, 0)),
            pl.BlockSpec((1, NLAB), lambda i: (0, 0)),
        ],
        out_specs=pl.BlockSpec((BT, NLAB), lambda i: (i, 0)),
        out_shape=jax.ShapeDtypeStruct((B, NLAB), jnp.float32),
    )(summed, lenf, W, bp)
    return out
